# baseline (speedup 1.0000x reference)
import functools

import jax
import jax.numpy as jnp
from jax import lax
from jax.experimental import pallas as pl
from jax.experimental.pallas import tpu as pltpu
from jax.experimental.pallas import tpu_sc as plsc

_MARGIN = 0.1
_IGNORE_INDEX = 0

_B = 1024
_V = 100000
_NC = 2
_NS = 16
_NW = _NC * _NS
_RPW = _B // _NW
_NBAND = _RPW // 8
_CW = 1408
_VA = 99968
_NCH = _VA // _CW
_T = _NBAND * _NCH
_NEG_INF = float("-inf")


def _sc_body(scores_hbm, gold_hbm, neg_out, golds_out,
             gold_v, buf_v, acc_v, neg_v, golds_v,
             gold_s, sems):
    wid = lax.axis_index("s") * _NC + lax.axis_index("c")
    base = wid * _RPW

    pltpu.sync_copy(gold_hbm.at[pl.ds(base, _RPW)], gold_v)
    for h in range(_RPW // 16):
        vec = gold_v[pl.ds(h * 16, 16)]
        for l in range(16):
            gold_s[h * 16 + l] = vec[l]

    zeros16 = jnp.zeros((16,), jnp.float32)
    for r in range(_RPW):
        golds_v[r] = zeros16

    def _start(t, bb):
        band = t // _NCH
        c = t % _NCH
        pltpu.async_copy(
            scores_hbm.at[pl.ds(base + band * 8, 8), pl.ds(c * _CW, _CW)],
            buf_v.at[bb], sems.at[bb],
        )

    def _wait(bb):
        pltpu.make_async_copy(
            scores_hbm.at[pl.ds(0, 8), pl.ds(0, _CW)], buf_v.at[bb], sems.at[bb]
        ).wait()

    _start(0, 0)

    @pl.loop(0, _T, step=2)
    def _chunks(t0):
        for bb in range(2):
            t = t0 + bb
            band = t // _NCH
            c = t % _NCH
            lo = c * _CW

            @pl.when(t + 1 < _T)
            def _prefetch():
                _start(t + 1, 1 - bb)

            _wait(bb)

            for r8 in range(8):
                r = band * 8 + r8
                g = gold_s[r]

                @pl.when((g >= lo) & (g < lo + _CW))
                def _patch(r8=r8, r=r, g=g):
                    idx = g - lo
                    al = (idx // 16) * 16
                    grp = buf_v[bb, r8, pl.ds(al, 16)]
                    sel = lax.iota(jnp.int32, 16) == idx - al
                    golds_v[r] = jnp.where(sel, grp, 0.0)
                    buf_v[bb, r8, pl.ds(al, 16)] = jnp.where(sel, _NEG_INF, grp)

            for r8 in range(8):
                r = band * 8 + r8
                init = jnp.where(c == 0, jnp.full((16,), _NEG_INF, jnp.float32),
                                 acc_v[r8])

                @pl.loop(0, _CW // 16, unroll=8, init_carry=init)
                def _cmax(i, m, r8=r8):
                    return jnp.maximum(m, buf_v[bb, r8, pl.ds(i * 16, 16)])

                acc_v[r8] = _cmax

                @pl.when(c == _NCH - 1)
                def _row_done(r8=r8, r=r):
                    neg_v[r] = acc_v[r8]

    pltpu.sync_copy(neg_v, neg_out.at[pl.ds(base, _RPW)])
    pltpu.sync_copy(golds_v, golds_out.at[pl.ds(base, _RPW)])


def _tail_kernel(x_ref, g_ref, negp_ref, goldp_ref, o_ref, *, b):
    x = x_ref[...]
    col = _VA + jax.lax.broadcasted_iota(jnp.int32, x.shape, 1)
    g = g_ref[...]
    is_gold = col == g
    invalid = col >= _V
    tail_neg = jnp.max(jnp.where(is_gold | invalid, -jnp.inf, x), axis=1,
                       keepdims=True)
    tail_gold = jnp.sum(jnp.where(is_gold, x, 0.0), axis=1, keepdims=True)
    neg = jnp.maximum(jnp.max(negp_ref[...], axis=1, keepdims=True), tail_neg)
    golds = jnp.sum(goldp_ref[...], axis=1, keepdims=True) + tail_gold
    loss = jnp.maximum(_MARGIN + neg - golds, 0.0)
    loss = loss * (g != _IGNORE_INDEX).astype(loss.dtype)
    o_ref[0, 0] = jnp.sum(loss) / b


@functools.partial(jax.jit, static_argnames=("interpret",))
def kernel(scores, gold, interpret=False):
    b, v = scores.shape
    gold32 = gold.astype(jnp.int32)
    neg_p, golds_p = pl.kernel(
        _sc_body,
        out_type=[
            jax.ShapeDtypeStruct((_B, 16), jnp.float32),
            jax.ShapeDtypeStruct((_B, 16), jnp.float32),
        ],
        mesh=plsc.VectorSubcoreMesh(core_axis_name="c", subcore_axis_name="s"),
        scratch_types=[
            pltpu.VMEM((_RPW,), jnp.int32),
            pltpu.VMEM((2, 8, _CW), jnp.float32),
            pltpu.VMEM((8, 16), jnp.float32),
            pltpu.VMEM((_RPW, 16), jnp.float32),
            pltpu.VMEM((_RPW, 16), jnp.float32),
            pltpu.SMEM((_RPW,), jnp.int32),
            pltpu.SemaphoreType.DMA((2,)),
        ],
        compiler_params=pltpu.CompilerParams(use_tc_tiling_on_sc=True),
        interpret=interpret,
    )(scores, gold32)

    out = pl.pallas_call(
        functools.partial(_tail_kernel, b=b),
        grid=(1,),
        in_specs=[
            pl.BlockSpec((b, 128), lambda i: (0, _VA // 128)),
            pl.BlockSpec((b, 1), lambda i: (0, 0)),
            pl.BlockSpec((b, 16), lambda i: (0, 0)),
            pl.BlockSpec((b, 16), lambda i: (0, 0)),
        ],
        out_specs=pl.BlockSpec(memory_space=pltpu.SMEM),
        out_shape=jax.ShapeDtypeStruct((1, 1), jnp.float32),
        interpret=interpret,
    )(scores, gold32.reshape(b, 1), neg_p, golds_p)
    return out[0, 0]


# speedup vs baseline: 4.6992x; 4.6992x over previous
import functools

import jax
import jax.numpy as jnp
from jax.experimental import pallas as pl
from jax.experimental.pallas import tpu as pltpu

_MARGIN = 0.1
_IGNORE_INDEX = 0


def _loss_kernel(x_ref, g_ref, o_ref, neg_acc, gold_acc, *, bh, v, b, nb):
    i = pl.program_id(0)

    @pl.when(i == 0)
    def _init():
        neg_acc[...] = jnp.full_like(neg_acc, -jnp.inf)
        gold_acc[...] = jnp.zeros_like(gold_acc)

    x = x_ref[...]
    row = i * bh + jax.lax.broadcasted_iota(jnp.int32, x.shape, 0)
    g = g_ref[...]
    is_gold = row == g
    neg = jnp.max(jnp.where(is_gold, -jnp.inf, x), axis=0, keepdims=True)
    neg_acc[...] = jnp.maximum(neg_acc[...], neg)
    gold_acc[...] += jnp.sum(jnp.where(is_gold, x, 0.0), axis=0, keepdims=True)

    @pl.when(i == nb - 1)
    def _final():
        loss = jnp.maximum(_MARGIN + neg_acc[...] - gold_acc[...], 0.0)
        loss = loss * (g != _IGNORE_INDEX).astype(loss.dtype)
        o_ref[0, 0] = jnp.sum(loss) / b


@functools.partial(jax.jit, static_argnames=("interpret",))
def kernel(scores, gold, interpret=False):
    b, v = scores.shape
    st = scores.T
    bh = 4000
    nb = pl.cdiv(v, bh)
    gold2 = gold.astype(jnp.int32).reshape(1, b)
    out = pl.pallas_call(
        functools.partial(_loss_kernel, bh=bh, v=v, b=b, nb=nb),
        grid=(nb,),
        in_specs=[
            pl.BlockSpec((bh, b), lambda i: (i, 0)),
            pl.BlockSpec((1, b), lambda i: (0, 0)),
        ],
        out_specs=pl.BlockSpec(memory_space=pltpu.SMEM),
        out_shape=jax.ShapeDtypeStruct((1, 1), jnp.float32),
        scratch_shapes=[
            pltpu.VMEM((1, b), jnp.float32),
            pltpu.VMEM((1, b), jnp.float32),
        ],
        interpret=interpret,
    )(st, gold2)
    return out[0, 0]
